# baseline (speedup 1.0000x reference)
import numpy as np
import jax
import jax.numpy as jnp
from jax import lax
from jax.experimental import pallas as pl
from jax.experimental.pallas import tpu as pltpu
from jax.experimental.pallas import tpu_sc as plsc

N = 10000
D = 128
DH = D // 2
DP = DH // 2
E = 320000
NC = 2
NS = 16
K = 128
CHUNKS = 158
PAIRS = CHUNKS // 2
EPW = CHUNKS * K
EPAD = EPW * NS
NPAD = 10240
RPT = NPAD // NS
BN = 400
f32 = jnp.float32

_P = np.array([32 * (i // 32)
               + (2 * (i % 32) if i % 32 < 16 else 2 * ((i % 32) - 16) + 1)
               for i in range(DH)])
_PFULL = np.concatenate([_P, DH + _P])


def _make_sc_agg(with_deg):
    mesh = plsc.VectorSubcoreMesh(
        core_axis_name="c", subcore_axis_name="s",
        num_cores=NC, num_subcores=NS)
    if with_deg:
        out_type = (jax.ShapeDtypeStruct((NC, NPAD, DH), f32),
                    jax.ShapeDtypeStruct((NS, NPAD), f32))
    else:
        out_type = jax.ShapeDtypeStruct((NC, NPAD, DH), f32)
    scratch = [
        pltpu.VMEM((CHUNKS, K), jnp.int32),
        pltpu.VMEM((CHUNKS, K), jnp.int32),
        pltpu.VMEM((K, DP), f32),
        pltpu.VMEM((K, DP), f32),
        pltpu.VMEM((K, DH), f32),
        pltpu.VMEM((K, DH), f32),
        pltpu.VMEM((NPAD,), f32),
        pltpu.VMEM_SHARED((NPAD, DH), f32),
        pltpu.SemaphoreType.DMA,
        pltpu.SemaphoreType.DMA,
        pltpu.SemaphoreType.DMA,
        pltpu.SemaphoreType.DMA,
    ]

    def body(xa_hbm, xb_hbm, src_hbm, dst_hbm, zrow_hbm, *rest):
        if with_deg:
            acc_out, deg_out = rest[0], rest[1]
            rest = rest[2:]
        else:
            acc_out = rest[0]
            rest = rest[1:]
        (src_v, dst_v, pa, pb, fa, fb, hist_v, acc_sh,
         sem_a, sem_b, sem_sa, sem_sb) = rest
        c = lax.axis_index("c")
        s = lax.axis_index("s")
        r0 = s * RPT
        pltpu.sync_copy(zrow_hbm, acc_sh.at[pl.ds(r0, RPT)])
        pltpu.sync_copy(src_hbm.at[s], src_v)
        pltpu.sync_copy(dst_hbm.at[s], dst_v)
        plsc.subcore_barrier()

        def gather(j, buf, sm):
            @pl.when(c == 0)
            def _():
                pltpu.async_copy(xa_hbm.at[src_v.at[j]], buf, sm)

            @pl.when(c == 1)
            def _():
                pltpu.async_copy(xb_hbm.at[src_v.at[j]], buf, sm)

        def gwait(buf, sm):
            pltpu.make_async_copy(xa_hbm.at[src_v.at[0]], buf, sm).wait()

        def unpack_chunk(pbuf, fbuf):
            for r in range(K):
                for m in range(DH // 32):
                    w16 = pbuf[r, pl.ds(16 * m, 16)]
                    b32 = plsc.bitcast(w16, jnp.bfloat16)
                    lo, hi = plsc.unpack(b32, format=plsc.PackFormat.INTERLEAVED)
                    fbuf[r, pl.ds(32 * m, 16)] = lo
                    fbuf[r, pl.ds(32 * m + 16, 16)] = hi

        gather(0, pa, sem_a)

        if with_deg:
            @pl.when(c == 0)
            def _():
                zero16 = jnp.zeros((16,), f32)
                one16 = jnp.full((16,), 1.0, f32)

                def zstep(i, carry):
                    hist_v[pl.ds(i * 16, 16)] = zero16
                    return carry

                lax.fori_loop(0, NPAD // 16, zstep, 0)

                def hstep(j, carry):
                    for l in range(K // 16):
                        idx16 = dst_v[j, pl.ds(l * 16, 16)]
                        plsc.addupdate_scatter(hist_v, [idx16], one16)
                    return carry

                lax.fori_loop(0, CHUNKS, hstep, 0)

        def pair(i, carry):
            j0 = 2 * i
            gather(j0 + 1, pb, sem_b)
            gwait(pa, sem_a)
            unpack_chunk(pa, fa)
            da = pltpu.async_copy(fa, acc_sh.at[dst_v.at[j0]], sem_sa,
                                  add=True)

            @pl.when(i < PAIRS - 1)
            def _():
                gather(j0 + 2, pa, sem_a)

            gwait(pb, sem_b)
            unpack_chunk(pb, fb)
            db = pltpu.async_copy(fb, acc_sh.at[dst_v.at[j0 + 1]], sem_sb,
                                  add=True)
            da.wait()
            db.wait()
            return carry

        lax.fori_loop(0, PAIRS, pair, 0)
        plsc.subcore_barrier()
        pltpu.sync_copy(acc_sh.at[pl.ds(r0, RPT)],
                        acc_out.at[c, pl.ds(r0, RPT)])
        if with_deg:
            @pl.when(c == 0)
            def _():
                pltpu.sync_copy(hist_v, deg_out.at[s])

    return pl.kernel(body, out_type=out_type, mesh=mesh,
                     scratch_types=scratch,
                     compiler_params=pltpu.CompilerParams(
                         use_tc_tiling_on_sc=False,
                         needs_layout_passes=False))


_sc_agg_deg = _make_sc_agg(True)
_sc_agg = _make_sc_agg(False)


def _make_tc_layer(split_out):
    def body(acc_ref, deg_ref, w_ref, b_ref, *out_refs):
        a = jnp.concatenate([acc_ref[0], acc_ref[1]], axis=1)
        dg = jnp.maximum(jnp.sum(deg_ref[...], axis=1), 1.0)[:, None]
        agg = a / dg
        y = lax.dot_general(agg, w_ref[...], (((1,), (1,)), ((), ())),
                            preferred_element_type=f32) + b_ref[...]
        y = jnp.where(y > 0.0, y, jnp.exp(y) - 1.0)
        if split_out:
            out_refs[0][...] = y[:, :DH].astype(jnp.bfloat16)
            out_refs[1][...] = y[:, DH:].astype(jnp.bfloat16)
        else:
            out_refs[0][...] = y

    if split_out:
        out_shape = (jax.ShapeDtypeStruct((N, DH), jnp.bfloat16),) * 2
        out_specs = (pl.BlockSpec((BN, DH), lambda i: (i, 0)),) * 2
    else:
        out_shape = jax.ShapeDtypeStruct((N, D), f32)
        out_specs = pl.BlockSpec((BN, D), lambda i: (i, 0))
    return pl.pallas_call(
        body,
        grid=(N // BN,),
        in_specs=[
            pl.BlockSpec((NC, BN, DH), lambda i: (0, i, 0)),
            pl.BlockSpec((BN, NS), lambda i: (i, 0)),
            pl.BlockSpec((D, D), lambda i: (0, 0)),
            pl.BlockSpec((1, D), lambda i: (0, 0)),
        ],
        out_specs=out_specs,
        out_shape=out_shape,
    )


_tc_mid = _make_tc_layer(True)
_tc_last = _make_tc_layer(False)


def _pack_cols(xb16):
    return lax.bitcast_convert_type(xb16.reshape(N, DP, 2), f32)


def kernel(h, edge_index, W1, b1, W2, b2):
    ei = edge_index.astype(jnp.int32)
    pad = EPAD - E
    src_p = jnp.concatenate(
        [ei[0], jnp.zeros((pad,), jnp.int32)]).reshape(NS, CHUNKS, K)
    dst_p = jnp.concatenate(
        [ei[1], jnp.full((pad,), NPAD - 1, jnp.int32)]).reshape(NS, CHUNKS, K)
    ha = _pack_cols(h[:, :DH].astype(jnp.bfloat16))
    hb = _pack_cols(h[:, DH:].astype(jnp.bfloat16))
    zrow = jnp.zeros((RPT, DH), f32)
    w1p = W1[:, _PFULL]
    w2p = W2[:, _PFULL]

    acc1, deg = _sc_agg_deg(ha, hb, src_p, dst_p, zrow)
    deg_t = deg.T
    x1a, x1b = _tc_mid(acc1, deg_t, w1p, b1.reshape(1, D))
    acc2 = _sc_agg(_pack_cols(x1a), _pack_cols(x1b), src_p, dst_p, zrow)
    return _tc_last(acc2, deg_t, w2p, b2.reshape(1, D))


# speedup vs baseline: 1.0596x; 1.0596x over previous
import numpy as np
import jax
import jax.numpy as jnp
from jax import lax
from jax.experimental import pallas as pl
from jax.experimental.pallas import tpu as pltpu
from jax.experimental.pallas import tpu_sc as plsc

N = 10000
D = 128
DH = D // 2
DP = DH // 2
E = 320000
NC = 2
NS = 16
K = 128
CHUNKS = 158
PAIRS = CHUNKS // 2
EPW = CHUNKS * K
EPAD = EPW * NS
NPAD = 10240
RPT = NPAD // NS
BN = 2000
f32 = jnp.float32

_P = np.array([32 * (i // 32)
               + (2 * (i % 32) if i % 32 < 16 else 2 * ((i % 32) - 16) + 1)
               for i in range(DH)])
_PFULL = np.concatenate([_P, DH + _P])


def _make_sc_agg(with_deg):
    mesh = plsc.VectorSubcoreMesh(
        core_axis_name="c", subcore_axis_name="s",
        num_cores=NC, num_subcores=NS)
    if with_deg:
        out_type = (jax.ShapeDtypeStruct((NC, NPAD, DH), f32),
                    jax.ShapeDtypeStruct((NS, NPAD), f32))
    else:
        out_type = jax.ShapeDtypeStruct((NC, NPAD, DH), f32)
    scratch = [
        pltpu.VMEM((CHUNKS, K), jnp.int32),
        pltpu.VMEM((CHUNKS, K), jnp.int32),
        pltpu.VMEM((K, DP), f32),
        pltpu.VMEM((K, DP), f32),
        pltpu.VMEM((K, DH), f32),
        pltpu.VMEM((K, DH), f32),
        pltpu.VMEM((NPAD,), f32),
        pltpu.VMEM_SHARED((NPAD, DH), f32),
        pltpu.SemaphoreType.DMA,
        pltpu.SemaphoreType.DMA,
        pltpu.SemaphoreType.DMA,
        pltpu.SemaphoreType.DMA,
    ]

    def body(xa_hbm, xb_hbm, src_hbm, dst_hbm, zrow_hbm, *rest):
        if with_deg:
            acc_out, deg_out = rest[0], rest[1]
            rest = rest[2:]
        else:
            acc_out = rest[0]
            rest = rest[1:]
        (src_v, dst_v, pa, pb, fa, fb, hist_v, acc_sh,
         sem_a, sem_b, sem_sa, sem_sb) = rest
        c = lax.axis_index("c")
        s = lax.axis_index("s")
        r0 = s * RPT
        pltpu.sync_copy(zrow_hbm, acc_sh.at[pl.ds(r0, RPT)])
        pltpu.sync_copy(src_hbm.at[s], src_v)
        pltpu.sync_copy(dst_hbm.at[s], dst_v)
        plsc.subcore_barrier()

        def gather(j, buf, sm):
            @pl.when(c == 0)
            def _():
                pltpu.async_copy(xa_hbm.at[src_v.at[j]], buf, sm)

            @pl.when(c == 1)
            def _():
                pltpu.async_copy(xb_hbm.at[src_v.at[j]], buf, sm)

        def gwait(buf, sm):
            pltpu.make_async_copy(xa_hbm.at[src_v.at[0]], buf, sm).wait()

        def unpack_chunk(pbuf, fbuf):
            for r in range(K):
                for m in range(DH // 32):
                    w16 = pbuf[r, pl.ds(16 * m, 16)]
                    b32 = plsc.bitcast(w16, jnp.bfloat16)
                    lo, hi = plsc.unpack(b32, format=plsc.PackFormat.INTERLEAVED)
                    fbuf[r, pl.ds(32 * m, 16)] = lo
                    fbuf[r, pl.ds(32 * m + 16, 16)] = hi

        gather(0, pa, sem_a)

        if with_deg:
            @pl.when(c == 0)
            def _():
                zero16 = jnp.zeros((16,), f32)
                one16 = jnp.full((16,), 1.0, f32)

                def zstep(i, carry):
                    hist_v[pl.ds(i * 16, 16)] = zero16
                    return carry

                lax.fori_loop(0, NPAD // 16, zstep, 0)

                def hstep(j, carry):
                    for l in range(K // 16):
                        idx16 = dst_v[j, pl.ds(l * 16, 16)]
                        plsc.addupdate_scatter(hist_v, [idx16], one16)
                    return carry

                lax.fori_loop(0, CHUNKS, hstep, 0)

        def pair(i, carry):
            j0 = 2 * i
            gather(j0 + 1, pb, sem_b)
            gwait(pa, sem_a)
            unpack_chunk(pa, fa)
            da = pltpu.async_copy(fa, acc_sh.at[dst_v.at[j0]], sem_sa,
                                  add=True)

            @pl.when(i < PAIRS - 1)
            def _():
                gather(j0 + 2, pa, sem_a)

            gwait(pb, sem_b)
            unpack_chunk(pb, fb)
            db = pltpu.async_copy(fb, acc_sh.at[dst_v.at[j0 + 1]], sem_sb,
                                  add=True)
            da.wait()
            db.wait()
            return carry

        lax.fori_loop(0, PAIRS, pair, 0)
        plsc.subcore_barrier()
        pltpu.sync_copy(acc_sh.at[pl.ds(r0, RPT)],
                        acc_out.at[c, pl.ds(r0, RPT)])
        if with_deg:
            @pl.when(c == 0)
            def _():
                pltpu.sync_copy(hist_v, deg_out.at[s])

    return pl.kernel(body, out_type=out_type, mesh=mesh,
                     scratch_types=scratch,
                     compiler_params=pltpu.CompilerParams(
                         use_tc_tiling_on_sc=False,
                         needs_layout_passes=False))


_sc_agg_deg = _make_sc_agg(True)
_sc_agg = _make_sc_agg(False)


def _make_tc_layer(split_out):
    def body(acc_ref, deg_ref, w_ref, b_ref, *out_refs):
        a = jnp.concatenate([acc_ref[0], acc_ref[1]], axis=1)
        dg = jnp.maximum(jnp.sum(deg_ref[...], axis=1), 1.0)[:, None]
        agg = a / dg
        y = lax.dot_general(agg, w_ref[...], (((1,), (1,)), ((), ())),
                            preferred_element_type=f32) + b_ref[...]
        y = jnp.where(y > 0.0, y, jnp.exp(y) - 1.0)
        if split_out:
            out_refs[0][...] = y[:, :DH].astype(jnp.bfloat16)
            out_refs[1][...] = y[:, DH:].astype(jnp.bfloat16)
        else:
            out_refs[0][...] = y

    if split_out:
        out_shape = (jax.ShapeDtypeStruct((N, DH), jnp.bfloat16),) * 2
        out_specs = (pl.BlockSpec((BN, DH), lambda i: (i, 0)),) * 2
    else:
        out_shape = jax.ShapeDtypeStruct((N, D), f32)
        out_specs = pl.BlockSpec((BN, D), lambda i: (i, 0))
    return pl.pallas_call(
        body,
        grid=(N // BN,),
        in_specs=[
            pl.BlockSpec((NC, BN, DH), lambda i: (0, i, 0)),
            pl.BlockSpec((BN, NS), lambda i: (i, 0)),
            pl.BlockSpec((D, D), lambda i: (0, 0)),
            pl.BlockSpec((1, D), lambda i: (0, 0)),
        ],
        out_specs=out_specs,
        out_shape=out_shape,
    )


_tc_mid = _make_tc_layer(True)
_tc_last = _make_tc_layer(False)


def _pack_cols(xb16):
    return lax.bitcast_convert_type(xb16.reshape(N, DP, 2), f32)


def kernel(h, edge_index, W1, b1, W2, b2):
    ei = edge_index.astype(jnp.int32)
    pad = EPAD - E
    src_p = jnp.concatenate(
        [ei[0], jnp.zeros((pad,), jnp.int32)]).reshape(NS, CHUNKS, K)
    dst_p = jnp.concatenate(
        [ei[1], jnp.full((pad,), NPAD - 1, jnp.int32)]).reshape(NS, CHUNKS, K)
    ha = _pack_cols(h[:, :DH].astype(jnp.bfloat16))
    hb = _pack_cols(h[:, DH:].astype(jnp.bfloat16))
    zrow = jnp.zeros((RPT, DH), f32)
    w1p = W1[:, _PFULL]
    w2p = W2[:, _PFULL]

    acc1, deg = _sc_agg_deg(ha, hb, src_p, dst_p, zrow)
    deg_t = deg.T
    x1a, x1b = _tc_mid(acc1, deg_t, w1p, b1.reshape(1, D))
    acc2 = _sc_agg(_pack_cols(x1a), _pack_cols(x1b), src_p, dst_p, zrow)
    return _tc_last(acc2, deg_t, w2p, b2.reshape(1, D))
